# baseline (speedup 1.0000x reference)
import functools

import jax
import jax.numpy as jnp
from jax import lax
from jax.experimental import pallas as pl
from jax.experimental.pallas import tpu as pltpu
from jax.experimental.pallas import tpu_sc as plsc

N = 10000
E = 320000
D = 128

NC, NS = 2, 16
NW = NC * NS
EPW = E // NW
CHUNK = 80
NCH = EPW // CHUNK
NPAD = 10240
PAD_PT = NPAD // NS
ROWS_PT = PAD_PT

ECHUNK = 80
ENCH = EPW // ECHUNK
ENBUF = 4
GOFF = 1
SOFF = 3
DNBUF = 5

_MESH = plsc.VectorSubcoreMesh(
    core_axis_name="c", subcore_axis_name="s", num_cores=NC, num_subcores=NS)


def _worker_id():
    return lax.axis_index("s") * NC + lax.axis_index("c")


@functools.partial(
    pl.kernel,
    out_type=[
        jax.ShapeDtypeStruct((NC, NPAD), jnp.float32),
        jax.ShapeDtypeStruct((NC, NPAD), jnp.float32),
    ],
    mesh=_MESH,
    scratch_types=(
        [pltpu.VMEM((CHUNK,), jnp.int32) for _ in range(DNBUF)]
        + [pltpu.VMEM((CHUNK,), jnp.int32) for _ in range(DNBUF)]
        + [pltpu.VMEM((CHUNK,), jnp.float32) for _ in range(DNBUF)]
        + [pltpu.VMEM((CHUNK,), jnp.float32),
           pltpu.VMEM_SHARED((NPAD,), jnp.float32),
           pltpu.VMEM_SHARED((NPAD,), jnp.float32),
           pltpu.SemaphoreType.DMA((DNBUF,)),
           pltpu.SemaphoreType.DMA((DNBUF,)),
           pltpu.SemaphoreType.DMA((DNBUF,))]
    ),
)
def _deg_kernel(row1, col1, z1, cnt_out, sl_out, *rest):
    rowf = rest[:DNBUF]
    colf = rest[DNBUF:2 * DNBUF]
    slf = rest[2 * DNBUF:3 * DNBUF]
    onesv, cnt_acc, sl_acc, isem, s1, s2 = rest[3 * DNBUF:]
    cid = lax.axis_index("c")
    sid = lax.axis_index("s")
    wid = _worker_id()
    base = wid * EPW
    pltpu.sync_copy(z1, cnt_acc.at[pl.ds(sid * PAD_PT, PAD_PT)])
    pltpu.sync_copy(z1, sl_acc.at[pl.ds(sid * PAD_PT, PAD_PT)])
    for i in range(CHUNK // 16):
        onesv[pl.ds(16 * i, 16)] = jnp.full((16,), 1.0, jnp.float32)
    plsc.subcore_barrier()

    @pl.loop(0, (NCH + 2 + DNBUF - 1) // DNBUF)
    def _outer(g):
        for b in range(DNBUF):
            j = g * DNBUF + b

            @pl.when((j >= DNBUF) & (j < NCH))
            def _reuse_wait():
                pltpu.make_async_copy(
                    onesv, cnt_acc.at[rowf[b]], s1.at[b]).wait()
                pltpu.make_async_copy(
                    slf[b], sl_acc.at[rowf[b]], s2.at[b]).wait()

            @pl.when(j < NCH)
            def _idx_load():
                off = base + j * CHUNK
                pltpu.async_copy(row1.at[pl.ds(off, CHUNK)], rowf[b],
                                 isem.at[b])
                pltpu.async_copy(col1.at[pl.ds(off, CHUNK)], colf[b],
                                 isem.at[b])

            b2 = (b - 2) % DNBUF
            j2 = j - 2

            @pl.when((j2 >= 0) & (j2 < NCH))
            def _scatter():
                off2 = base + j2 * CHUNK
                pltpu.make_async_copy(row1.at[pl.ds(off2, CHUNK)], rowf[b2],
                                      isem.at[b2]).wait()
                pltpu.make_async_copy(col1.at[pl.ds(off2, CHUNK)], colf[b2],
                                      isem.at[b2]).wait()
                for i in range(CHUNK // 16):
                    r = rowf[b2][pl.ds(16 * i, 16)]
                    c = colf[b2][pl.ds(16 * i, 16)]
                    slf[b2][pl.ds(16 * i, 16)] = jnp.where(
                        r == c, jnp.full((16,), 1.0, jnp.float32),
                        jnp.zeros((16,), jnp.float32))
                pltpu.async_copy(onesv, cnt_acc.at[rowf[b2]], s1.at[b2],
                                 add=True)
                pltpu.async_copy(slf[b2], sl_acc.at[rowf[b2]], s2.at[b2],
                                 add=True)

    for b in range(DNBUF):
        pltpu.make_async_copy(onesv, cnt_acc.at[rowf[b]], s1.at[b]).wait()
        pltpu.make_async_copy(slf[b], sl_acc.at[rowf[b]], s2.at[b]).wait()
    plsc.subcore_barrier()
    pltpu.sync_copy(cnt_acc.at[pl.ds(sid * PAD_PT, PAD_PT)],
                    cnt_out.at[cid, pl.ds(sid * PAD_PT, PAD_PT)])
    pltpu.sync_copy(sl_acc.at[pl.ds(sid * PAD_PT, PAD_PT)],
                    sl_out.at[cid, pl.ds(sid * PAD_PT, PAD_PT)])


@functools.partial(
    pl.kernel,
    out_type=jax.ShapeDtypeStruct((NC, NPAD, D), jnp.float32),
    mesh=_MESH,
    scratch_types=(
        [pltpu.VMEM((ECHUNK,), jnp.int32) for _ in range(ENBUF)]
        + [pltpu.VMEM((ECHUNK,), jnp.int32) for _ in range(ENBUF)]
        + [pltpu.VMEM((ECHUNK, D), jnp.float32) for _ in range(ENBUF)]
        + [pltpu.VMEM_SHARED((NPAD, D), jnp.float32),
           pltpu.SemaphoreType.DMA((ENBUF,)),
           pltpu.SemaphoreType.DMA((ENBUF,)),
           pltpu.SemaphoreType.DMA((ENBUF,))]
    ),
)
def _edge_kernel(xp, row1, col1, z2, out, *rest):
    rowf = rest[:ENBUF]
    colf = rest[ENBUF:2 * ENBUF]
    xbs = rest[2 * ENBUF:3 * ENBUF]
    acc, isem, gsem, ssem = rest[3 * ENBUF:]
    cid = lax.axis_index("c")
    sid = lax.axis_index("s")
    wid = _worker_id()
    base = wid * EPW
    pltpu.sync_copy(z2, acc.at[pl.ds(sid * ROWS_PT, ROWS_PT)])
    plsc.subcore_barrier()

    @pl.loop(0, (ENCH + SOFF + ENBUF - 1) // ENBUF)
    def _outer(g):
        for b in range(ENBUF):
            j = g * ENBUF + b

            @pl.when((j >= ENBUF) & (j < ENCH))
            def _reuse_wait():
                pltpu.make_async_copy(
                    xbs[b], acc.at[colf[b]], ssem.at[b]).wait()

            @pl.when(j < ENCH)
            def _idx_load():
                off = base + j * ECHUNK
                pltpu.async_copy(row1.at[pl.ds(off, ECHUNK)], rowf[b],
                                 isem.at[b])
                pltpu.async_copy(col1.at[pl.ds(off, ECHUNK)], colf[b],
                                 isem.at[b])

            b2 = (b - GOFF) % ENBUF
            j2 = j - GOFF

            @pl.when((j2 >= 0) & (j2 < ENCH))
            def _gather():
                off2 = base + j2 * ECHUNK
                pltpu.make_async_copy(row1.at[pl.ds(off2, ECHUNK)], rowf[b2],
                                      isem.at[b2]).wait()
                pltpu.make_async_copy(col1.at[pl.ds(off2, ECHUNK)], colf[b2],
                                      isem.at[b2]).wait()
                pltpu.async_copy(xp.at[rowf[b2]], xbs[b2], gsem.at[b2])

            b3 = (b - SOFF) % ENBUF
            j3 = j - SOFF

            @pl.when((j3 >= 0) & (j3 < ENCH))
            def _scatter():
                pltpu.make_async_copy(xp.at[rowf[b3]], xbs[b3],
                                      gsem.at[b3]).wait()
                pltpu.async_copy(xbs[b3], acc.at[colf[b3]], ssem.at[b3],
                                 add=True)

    for b in range(ENBUF):
        pltpu.make_async_copy(xbs[b], acc.at[colf[b]], ssem.at[b]).wait()
    plsc.subcore_barrier()
    pltpu.sync_copy(acc.at[pl.ds(sid * ROWS_PT, ROWS_PT)],
                    out.at[cid, pl.ds(sid * ROWS_PT, ROWS_PT)])


BLK = 1000


def _prep_body(cnt_ref, sl_ref, x_ref, dc_ref, xp_ref):
    cnt = cnt_ref[0] + cnt_ref[1]
    sl = sl_ref[0] + sl_ref[1]
    deg = cnt - sl
    dis = jnp.where(deg > 0, lax.rsqrt(jnp.maximum(deg, 1e-12)), 0.0)
    corr = sl * dis * dis
    dc_ref[...] = jnp.concatenate(
        [dis, corr, jnp.zeros((BLK, D - 2), jnp.float32)], axis=1)
    xp_ref[...] = dis * x_ref[...]


_prep_call = pl.pallas_call(
    _prep_body,
    grid=(N // BLK,),
    in_specs=[
        pl.BlockSpec((NC, BLK, 1), lambda i: (0, i, 0)),
        pl.BlockSpec((NC, BLK, 1), lambda i: (0, i, 0)),
        pl.BlockSpec((BLK, D), lambda i: (i, 0)),
    ],
    out_specs=[
        pl.BlockSpec((BLK, D), lambda i: (i, 0)),
        pl.BlockSpec((BLK, D), lambda i: (i, 0)),
    ],
    out_shape=[
        jax.ShapeDtypeStruct((N, D), jnp.float32),
        jax.ShapeDtypeStruct((N, D), jnp.float32),
    ],
)


def _dense_body(x_ref, acc_ref, dc_ref, w0_ref, w1_ref, b_ref,
                *out_refs, relu):
    x = x_ref[...]
    dis = dc_ref[:, 0:1]
    tx = dc_ref[:, 1:2] * x - dis * (acc_ref[0] + acc_ref[1])
    y = (jnp.dot(x, w0_ref[...], preferred_element_type=jnp.float32)
         + jnp.dot(tx, w1_ref[...], preferred_element_type=jnp.float32)
         + b_ref[...])
    if relu:
        y = jnp.maximum(y, 0.0)
        out_refs[0][...] = y
        out_refs[1][...] = dis * y
    else:
        out_refs[0][...] = y


def _make_dense(relu):
    n_out = 2 if relu else 1
    return pl.pallas_call(
        functools.partial(_dense_body, relu=relu),
        grid=(N // BLK,),
        in_specs=[
            pl.BlockSpec((BLK, D), lambda i: (i, 0)),
            pl.BlockSpec((NC, BLK, D), lambda i: (0, i, 0)),
            pl.BlockSpec((BLK, D), lambda i: (i, 0)),
            pl.BlockSpec((D, D), lambda i: (0, 0)),
            pl.BlockSpec((D, D), lambda i: (0, 0)),
            pl.BlockSpec((1, D), lambda i: (0, 0)),
        ],
        out_specs=[pl.BlockSpec((BLK, D), lambda i: (i, 0))] * n_out,
        out_shape=[jax.ShapeDtypeStruct((N, D), jnp.float32)] * n_out,
    )


_dense_relu = _make_dense(True)
_dense_last = _make_dense(False)


def kernel(embedding, W0a, W1a, b1, W0b, W1b, b2, prop_edge_index):
    pei = prop_edge_index.astype(jnp.int32)
    row1, col1 = pei[0], pei[1]
    z1 = jnp.zeros((PAD_PT,), jnp.float32)
    z2 = jnp.zeros((ROWS_PT, D), jnp.float32)

    cnt_p, sl_p = _deg_kernel(row1, col1, z1)
    dc, xp1 = _prep_call(cnt_p[:, :N, None], sl_p[:, :N, None], embedding)
    acc1 = _edge_kernel(xp1, row1, col1, z2)
    h, xp2 = _dense_relu(embedding, acc1, dc, W0a, W1a, b1.reshape(1, D))
    acc2 = _edge_kernel(xp2, row1, col1, z2)
    out, = _dense_last(h, acc2, dc, W0b, W1b, b2.reshape(1, D))
    return out


# speedup vs baseline: 1.0031x; 1.0031x over previous
import functools

import jax
import jax.numpy as jnp
from jax import lax
from jax.experimental import pallas as pl
from jax.experimental.pallas import tpu as pltpu
from jax.experimental.pallas import tpu_sc as plsc

N = 10000
E = 320000
D = 128

NC, NS = 2, 16
NW = NC * NS
EPW = E // NW
CHUNK = 80
NCH = EPW // CHUNK
NPAD = 10240
PAD_PT = NPAD // NS
ROWS_PT = PAD_PT

ECHUNK = 80
ENCH = EPW // ECHUNK
ENBUF = 4
GOFF = 1
SOFF = 3
DNBUF = 5

_MESH = plsc.VectorSubcoreMesh(
    core_axis_name="c", subcore_axis_name="s", num_cores=NC, num_subcores=NS)


def _worker_id():
    return lax.axis_index("s") * NC + lax.axis_index("c")


@functools.partial(
    pl.kernel,
    out_type=jax.ShapeDtypeStruct((2, NC, NPAD), jnp.float32),
    mesh=_MESH,
    scratch_types=(
        [pltpu.VMEM((CHUNK,), jnp.int32) for _ in range(DNBUF)]
        + [pltpu.VMEM((CHUNK,), jnp.int32) for _ in range(DNBUF)]
        + [pltpu.VMEM((CHUNK,), jnp.float32) for _ in range(DNBUF)]
        + [pltpu.VMEM((CHUNK,), jnp.float32),
           pltpu.VMEM_SHARED((NPAD,), jnp.float32),
           pltpu.VMEM_SHARED((NPAD,), jnp.float32),
           pltpu.SemaphoreType.DMA((DNBUF,)),
           pltpu.SemaphoreType.DMA((DNBUF,)),
           pltpu.SemaphoreType.DMA((DNBUF,))]
    ),
)
def _deg_kernel(row1, col1, z1, cs_out, *rest):
    rowf = rest[:DNBUF]
    colf = rest[DNBUF:2 * DNBUF]
    slf = rest[2 * DNBUF:3 * DNBUF]
    onesv, cnt_acc, sl_acc, isem, s1, s2 = rest[3 * DNBUF:]
    cid = lax.axis_index("c")
    sid = lax.axis_index("s")
    wid = _worker_id()
    base = wid * EPW
    pltpu.sync_copy(z1, cnt_acc.at[pl.ds(sid * PAD_PT, PAD_PT)])
    pltpu.sync_copy(z1, sl_acc.at[pl.ds(sid * PAD_PT, PAD_PT)])
    for i in range(CHUNK // 16):
        onesv[pl.ds(16 * i, 16)] = jnp.full((16,), 1.0, jnp.float32)
    plsc.subcore_barrier()

    @pl.loop(0, (NCH + 2 + DNBUF - 1) // DNBUF)
    def _outer(g):
        for b in range(DNBUF):
            j = g * DNBUF + b

            @pl.when((j >= DNBUF) & (j < NCH))
            def _reuse_wait():
                pltpu.make_async_copy(
                    onesv, cnt_acc.at[rowf[b]], s1.at[b]).wait()
                pltpu.make_async_copy(
                    slf[b], sl_acc.at[rowf[b]], s2.at[b]).wait()

            @pl.when(j < NCH)
            def _idx_load():
                off = base + j * CHUNK
                pltpu.async_copy(row1.at[pl.ds(off, CHUNK)], rowf[b],
                                 isem.at[b])
                pltpu.async_copy(col1.at[pl.ds(off, CHUNK)], colf[b],
                                 isem.at[b])

            b2 = (b - 2) % DNBUF
            j2 = j - 2

            @pl.when((j2 >= 0) & (j2 < NCH))
            def _scatter():
                off2 = base + j2 * CHUNK
                pltpu.make_async_copy(row1.at[pl.ds(off2, CHUNK)], rowf[b2],
                                      isem.at[b2]).wait()
                pltpu.make_async_copy(col1.at[pl.ds(off2, CHUNK)], colf[b2],
                                      isem.at[b2]).wait()
                for i in range(CHUNK // 16):
                    r = rowf[b2][pl.ds(16 * i, 16)]
                    c = colf[b2][pl.ds(16 * i, 16)]
                    slf[b2][pl.ds(16 * i, 16)] = jnp.where(
                        r == c, jnp.full((16,), 1.0, jnp.float32),
                        jnp.zeros((16,), jnp.float32))
                pltpu.async_copy(onesv, cnt_acc.at[rowf[b2]], s1.at[b2],
                                 add=True)
                pltpu.async_copy(slf[b2], sl_acc.at[rowf[b2]], s2.at[b2],
                                 add=True)

    for b in range(DNBUF):
        pltpu.make_async_copy(onesv, cnt_acc.at[rowf[b]], s1.at[b]).wait()
        pltpu.make_async_copy(slf[b], sl_acc.at[rowf[b]], s2.at[b]).wait()
    plsc.subcore_barrier()
    pltpu.sync_copy(cnt_acc.at[pl.ds(sid * PAD_PT, PAD_PT)],
                    cs_out.at[0, cid, pl.ds(sid * PAD_PT, PAD_PT)])
    pltpu.sync_copy(sl_acc.at[pl.ds(sid * PAD_PT, PAD_PT)],
                    cs_out.at[1, cid, pl.ds(sid * PAD_PT, PAD_PT)])


@functools.partial(
    pl.kernel,
    out_type=jax.ShapeDtypeStruct((NC, NPAD, D), jnp.float32),
    mesh=_MESH,
    scratch_types=(
        [pltpu.VMEM((ECHUNK,), jnp.int32) for _ in range(ENBUF)]
        + [pltpu.VMEM((ECHUNK,), jnp.int32) for _ in range(ENBUF)]
        + [pltpu.VMEM((ECHUNK, D), jnp.float32) for _ in range(ENBUF)]
        + [pltpu.VMEM_SHARED((NPAD, D), jnp.float32),
           pltpu.SemaphoreType.DMA((ENBUF,)),
           pltpu.SemaphoreType.DMA((ENBUF,)),
           pltpu.SemaphoreType.DMA((ENBUF,))]
    ),
)
def _edge_kernel(xp, row1, col1, z2, out, *rest):
    rowf = rest[:ENBUF]
    colf = rest[ENBUF:2 * ENBUF]
    xbs = rest[2 * ENBUF:3 * ENBUF]
    acc, isem, gsem, ssem = rest[3 * ENBUF:]
    cid = lax.axis_index("c")
    sid = lax.axis_index("s")
    wid = _worker_id()
    base = wid * EPW
    pltpu.sync_copy(z2, acc.at[pl.ds(sid * ROWS_PT, ROWS_PT)])
    plsc.subcore_barrier()

    @pl.loop(0, (ENCH + SOFF + ENBUF - 1) // ENBUF)
    def _outer(g):
        for b in range(ENBUF):
            j = g * ENBUF + b

            @pl.when((j >= ENBUF) & (j < ENCH))
            def _reuse_wait():
                pltpu.make_async_copy(
                    xbs[b], acc.at[colf[b]], ssem.at[b]).wait()

            @pl.when(j < ENCH)
            def _idx_load():
                off = base + j * ECHUNK
                pltpu.async_copy(row1.at[pl.ds(off, ECHUNK)], rowf[b],
                                 isem.at[b])
                pltpu.async_copy(col1.at[pl.ds(off, ECHUNK)], colf[b],
                                 isem.at[b])

            b2 = (b - GOFF) % ENBUF
            j2 = j - GOFF

            @pl.when((j2 >= 0) & (j2 < ENCH))
            def _gather():
                off2 = base + j2 * ECHUNK
                pltpu.make_async_copy(row1.at[pl.ds(off2, ECHUNK)], rowf[b2],
                                      isem.at[b2]).wait()
                pltpu.make_async_copy(col1.at[pl.ds(off2, ECHUNK)], colf[b2],
                                      isem.at[b2]).wait()
                pltpu.async_copy(xp.at[rowf[b2]], xbs[b2], gsem.at[b2])

            b3 = (b - SOFF) % ENBUF
            j3 = j - SOFF

            @pl.when((j3 >= 0) & (j3 < ENCH))
            def _scatter():
                pltpu.make_async_copy(xp.at[rowf[b3]], xbs[b3],
                                      gsem.at[b3]).wait()
                pltpu.async_copy(xbs[b3], acc.at[colf[b3]], ssem.at[b3],
                                 add=True)

    for b in range(ENBUF):
        pltpu.make_async_copy(xbs[b], acc.at[colf[b]], ssem.at[b]).wait()
    plsc.subcore_barrier()
    pltpu.sync_copy(acc.at[pl.ds(sid * ROWS_PT, ROWS_PT)],
                    out.at[cid, pl.ds(sid * ROWS_PT, ROWS_PT)])


BLK = 1000


def _prep_body(cs_ref, x_ref, dc_ref, xp_ref):
    cnt = cs_ref[0, 0] + cs_ref[0, 1]
    sl = cs_ref[1, 0] + cs_ref[1, 1]
    deg = cnt - sl
    dis = jnp.where(deg > 0, lax.rsqrt(jnp.maximum(deg, 1e-12)), 0.0)
    corr = sl * dis * dis
    dc_ref[...] = jnp.concatenate(
        [dis, corr, jnp.zeros((BLK, D - 2), jnp.float32)], axis=1)
    xp_ref[...] = dis * x_ref[...]


_prep_call = pl.pallas_call(
    _prep_body,
    grid=(N // BLK,),
    in_specs=[
        pl.BlockSpec((2, NC, BLK, 1), lambda i: (0, 0, i, 0)),
        pl.BlockSpec((BLK, D), lambda i: (i, 0)),
    ],
    out_specs=[
        pl.BlockSpec((BLK, D), lambda i: (i, 0)),
        pl.BlockSpec((BLK, D), lambda i: (i, 0)),
    ],
    out_shape=[
        jax.ShapeDtypeStruct((N, D), jnp.float32),
        jax.ShapeDtypeStruct((N, D), jnp.float32),
    ],
)


def _dense_body(x_ref, acc_ref, dc_ref, w0_ref, w1_ref, b_ref,
                *out_refs, relu):
    x = x_ref[...]
    dis = dc_ref[:, 0:1]
    tx = dc_ref[:, 1:2] * x - dis * (acc_ref[0] + acc_ref[1])
    y = (jnp.dot(x, w0_ref[...], preferred_element_type=jnp.float32)
         + jnp.dot(tx, w1_ref[...], preferred_element_type=jnp.float32)
         + b_ref[...])
    if relu:
        y = jnp.maximum(y, 0.0)
        out_refs[0][...] = y
        out_refs[1][...] = dis * y
    else:
        out_refs[0][...] = y


def _make_dense(relu):
    n_out = 2 if relu else 1
    return pl.pallas_call(
        functools.partial(_dense_body, relu=relu),
        grid=(N // BLK,),
        in_specs=[
            pl.BlockSpec((BLK, D), lambda i: (i, 0)),
            pl.BlockSpec((NC, BLK, D), lambda i: (0, i, 0)),
            pl.BlockSpec((BLK, D), lambda i: (i, 0)),
            pl.BlockSpec((D, D), lambda i: (0, 0)),
            pl.BlockSpec((D, D), lambda i: (0, 0)),
            pl.BlockSpec((1, D), lambda i: (0, 0)),
        ],
        out_specs=[pl.BlockSpec((BLK, D), lambda i: (i, 0))] * n_out,
        out_shape=[jax.ShapeDtypeStruct((N, D), jnp.float32)] * n_out,
    )


_dense_relu = _make_dense(True)
_dense_last = _make_dense(False)


def kernel(embedding, W0a, W1a, b1, W0b, W1b, b2, prop_edge_index):
    pei = prop_edge_index.astype(jnp.int32)
    row1, col1 = pei[0], pei[1]
    z1 = jnp.zeros((PAD_PT,), jnp.float32)
    z2 = jnp.zeros((ROWS_PT, D), jnp.float32)

    cs_p = _deg_kernel(row1, col1, z1)
    dc, xp1 = _prep_call(cs_p[:, :, :N, None], embedding)
    acc1 = _edge_kernel(xp1, row1, col1, z2)
    h, xp2 = _dense_relu(embedding, acc1, dc, W0a, W1a, b1.reshape(1, D))
    acc2 = _edge_kernel(xp2, row1, col1, z2)
    out, = _dense_last(h, acc2, dc, W0b, W1b, b2.reshape(1, D))
    return out


# speedup vs baseline: 1.0208x; 1.0177x over previous
import functools

import jax
import jax.numpy as jnp
from jax import lax
from jax.experimental import pallas as pl
from jax.experimental.pallas import tpu as pltpu
from jax.experimental.pallas import tpu_sc as plsc

N = 10000
E = 320000
D = 128

NC, NS = 2, 16
NW = NC * NS
EPW = E // NW
CHUNK = 80
NCH = EPW // CHUNK
NPAD = 10240
PAD_PT = NPAD // NS
ROWS_PT = PAD_PT

ECHUNK = 80
ENCH = EPW // ECHUNK
ENBUF = 4
GOFF = 1
SOFF = 3
DNBUF = 5
DCHUNK = 128
NCHD = EPW // DCHUNK
DTAIL = EPW - NCHD * DCHUNK

_MESH = plsc.VectorSubcoreMesh(
    core_axis_name="c", subcore_axis_name="s", num_cores=NC, num_subcores=NS)


def _worker_id():
    return lax.axis_index("s") * NC + lax.axis_index("c")


@functools.partial(
    pl.kernel,
    out_type=jax.ShapeDtypeStruct((2, NC, NPAD), jnp.float32),
    mesh=_MESH,
    scratch_types=(
        [pltpu.VMEM((DCHUNK,), jnp.int32) for _ in range(DNBUF)]
        + [pltpu.VMEM((DCHUNK,), jnp.int32) for _ in range(DNBUF)]
        + [pltpu.VMEM((DCHUNK,), jnp.float32) for _ in range(DNBUF)]
        + [pltpu.VMEM((DTAIL,), jnp.int32),
           pltpu.VMEM((DTAIL,), jnp.int32),
           pltpu.VMEM((DTAIL,), jnp.float32),
           pltpu.VMEM((DCHUNK,), jnp.float32),
           pltpu.VMEM_SHARED((NPAD,), jnp.float32),
           pltpu.VMEM_SHARED((NPAD,), jnp.float32),
           pltpu.SemaphoreType.DMA((DNBUF,)),
           pltpu.SemaphoreType.DMA((DNBUF,)),
           pltpu.SemaphoreType.DMA((DNBUF,))]
    ),
)
def _deg_kernel(row1, col1, z1, cs_out, *rest):
    rowf = rest[:DNBUF]
    colf = rest[DNBUF:2 * DNBUF]
    slf = rest[2 * DNBUF:3 * DNBUF]
    rowt, colt, slt, onesv, cnt_acc, sl_acc, isem, s1, s2 = rest[3 * DNBUF:]
    cid = lax.axis_index("c")
    sid = lax.axis_index("s")
    wid = _worker_id()
    base = wid * EPW
    pltpu.sync_copy(z1, cnt_acc.at[pl.ds(sid * PAD_PT, PAD_PT)])
    pltpu.sync_copy(z1, sl_acc.at[pl.ds(sid * PAD_PT, PAD_PT)])
    for i in range(DCHUNK // 16):
        onesv[pl.ds(16 * i, 16)] = jnp.full((16,), 1.0, jnp.float32)
    plsc.subcore_barrier()

    @pl.loop(0, (NCHD + 2 + DNBUF - 1) // DNBUF)
    def _outer(g):
        for b in range(DNBUF):
            j = g * DNBUF + b

            @pl.when((j >= DNBUF) & (j < NCHD))
            def _reuse_wait():
                pltpu.make_async_copy(
                    onesv, cnt_acc.at[rowf[b]], s1.at[b]).wait()
                pltpu.make_async_copy(
                    slf[b], sl_acc.at[rowf[b]], s2.at[b]).wait()

            @pl.when(j < NCHD)
            def _idx_load():
                off = base + j * DCHUNK
                pltpu.async_copy(row1.at[pl.ds(off, DCHUNK)], rowf[b],
                                 isem.at[b])
                pltpu.async_copy(col1.at[pl.ds(off, DCHUNK)], colf[b],
                                 isem.at[b])

            b2 = (b - 2) % DNBUF
            j2 = j - 2

            @pl.when((j2 >= 0) & (j2 < NCHD))
            def _scatter():
                off2 = base + j2 * DCHUNK
                pltpu.make_async_copy(row1.at[pl.ds(off2, DCHUNK)], rowf[b2],
                                      isem.at[b2]).wait()
                pltpu.make_async_copy(col1.at[pl.ds(off2, DCHUNK)], colf[b2],
                                      isem.at[b2]).wait()
                for i in range(DCHUNK // 16):
                    r = rowf[b2][pl.ds(16 * i, 16)]
                    c = colf[b2][pl.ds(16 * i, 16)]
                    slf[b2][pl.ds(16 * i, 16)] = jnp.where(
                        r == c, jnp.full((16,), 1.0, jnp.float32),
                        jnp.zeros((16,), jnp.float32))
                pltpu.async_copy(onesv, cnt_acc.at[rowf[b2]], s1.at[b2],
                                 add=True)
                pltpu.async_copy(slf[b2], sl_acc.at[rowf[b2]], s2.at[b2],
                                 add=True)

    toff = base + NCHD * DCHUNK
    pltpu.sync_copy(row1.at[pl.ds(toff, DTAIL)], rowt)
    pltpu.sync_copy(col1.at[pl.ds(toff, DTAIL)], colt)
    for i in range(DTAIL // 16):
        r = rowt[pl.ds(16 * i, 16)]
        c = colt[pl.ds(16 * i, 16)]
        slt[pl.ds(16 * i, 16)] = jnp.where(
            r == c, jnp.full((16,), 1.0, jnp.float32),
            jnp.zeros((16,), jnp.float32))
    pltpu.sync_copy(onesv.at[pl.ds(0, DTAIL)], cnt_acc.at[rowt], add=True)
    pltpu.sync_copy(slt, sl_acc.at[rowt], add=True)
    for b in range(DNBUF):
        pltpu.make_async_copy(onesv, cnt_acc.at[rowf[b]], s1.at[b]).wait()
        pltpu.make_async_copy(slf[b], sl_acc.at[rowf[b]], s2.at[b]).wait()
    plsc.subcore_barrier()
    pltpu.sync_copy(cnt_acc.at[pl.ds(sid * PAD_PT, PAD_PT)],
                    cs_out.at[0, cid, pl.ds(sid * PAD_PT, PAD_PT)])
    pltpu.sync_copy(sl_acc.at[pl.ds(sid * PAD_PT, PAD_PT)],
                    cs_out.at[1, cid, pl.ds(sid * PAD_PT, PAD_PT)])


@functools.partial(
    pl.kernel,
    out_type=jax.ShapeDtypeStruct((NC, NPAD, D), jnp.float32),
    mesh=_MESH,
    scratch_types=(
        [pltpu.VMEM((ECHUNK,), jnp.int32) for _ in range(ENBUF)]
        + [pltpu.VMEM((ECHUNK,), jnp.int32) for _ in range(ENBUF)]
        + [pltpu.VMEM((ECHUNK, D), jnp.float32) for _ in range(ENBUF)]
        + [pltpu.VMEM_SHARED((NPAD, D), jnp.float32),
           pltpu.SemaphoreType.DMA((ENBUF,)),
           pltpu.SemaphoreType.DMA((ENBUF,)),
           pltpu.SemaphoreType.DMA((ENBUF,))]
    ),
)
def _edge_kernel(xp, row1, col1, z2, out, *rest):
    rowf = rest[:ENBUF]
    colf = rest[ENBUF:2 * ENBUF]
    xbs = rest[2 * ENBUF:3 * ENBUF]
    acc, isem, gsem, ssem = rest[3 * ENBUF:]
    cid = lax.axis_index("c")
    sid = lax.axis_index("s")
    wid = _worker_id()
    base = wid * EPW
    pltpu.sync_copy(z2, acc.at[pl.ds(sid * ROWS_PT, ROWS_PT)])
    plsc.subcore_barrier()

    @pl.loop(0, (ENCH + SOFF + ENBUF - 1) // ENBUF)
    def _outer(g):
        for b in range(ENBUF):
            j = g * ENBUF + b

            @pl.when((j >= ENBUF) & (j < ENCH))
            def _reuse_wait():
                pltpu.make_async_copy(
                    xbs[b], acc.at[colf[b]], ssem.at[b]).wait()

            @pl.when(j < ENCH)
            def _idx_load():
                off = base + j * ECHUNK
                pltpu.async_copy(row1.at[pl.ds(off, ECHUNK)], rowf[b],
                                 isem.at[b])
                pltpu.async_copy(col1.at[pl.ds(off, ECHUNK)], colf[b],
                                 isem.at[b])

            b2 = (b - GOFF) % ENBUF
            j2 = j - GOFF

            @pl.when((j2 >= 0) & (j2 < ENCH))
            def _gather():
                off2 = base + j2 * ECHUNK
                pltpu.make_async_copy(row1.at[pl.ds(off2, ECHUNK)], rowf[b2],
                                      isem.at[b2]).wait()
                pltpu.make_async_copy(col1.at[pl.ds(off2, ECHUNK)], colf[b2],
                                      isem.at[b2]).wait()
                pltpu.async_copy(xp.at[rowf[b2]], xbs[b2], gsem.at[b2])

            b3 = (b - SOFF) % ENBUF
            j3 = j - SOFF

            @pl.when((j3 >= 0) & (j3 < ENCH))
            def _scatter():
                pltpu.make_async_copy(xp.at[rowf[b3]], xbs[b3],
                                      gsem.at[b3]).wait()
                pltpu.async_copy(xbs[b3], acc.at[colf[b3]], ssem.at[b3],
                                 add=True)

    for b in range(ENBUF):
        pltpu.make_async_copy(xbs[b], acc.at[colf[b]], ssem.at[b]).wait()
    plsc.subcore_barrier()
    pltpu.sync_copy(acc.at[pl.ds(sid * ROWS_PT, ROWS_PT)],
                    out.at[cid, pl.ds(sid * ROWS_PT, ROWS_PT)])


BLK = 1000


def _prep_body(cs_ref, x_ref, dc_ref, xp_ref):
    cnt = cs_ref[0, 0] + cs_ref[0, 1]
    sl = cs_ref[1, 0] + cs_ref[1, 1]
    deg = cnt - sl
    dis = jnp.where(deg > 0, lax.rsqrt(jnp.maximum(deg, 1e-12)), 0.0)
    corr = sl * dis * dis
    dc_ref[...] = jnp.concatenate(
        [dis, corr, jnp.zeros((BLK, D - 2), jnp.float32)], axis=1)
    xp_ref[...] = dis * x_ref[...]


_prep_call = pl.pallas_call(
    _prep_body,
    grid=(N // BLK,),
    in_specs=[
        pl.BlockSpec((2, NC, BLK, 1), lambda i: (0, 0, i, 0)),
        pl.BlockSpec((BLK, D), lambda i: (i, 0)),
    ],
    out_specs=[
        pl.BlockSpec((BLK, D), lambda i: (i, 0)),
        pl.BlockSpec((BLK, D), lambda i: (i, 0)),
    ],
    out_shape=[
        jax.ShapeDtypeStruct((N, D), jnp.float32),
        jax.ShapeDtypeStruct((N, D), jnp.float32),
    ],
)


def _dense_body(x_ref, acc_ref, dc_ref, w0_ref, w1_ref, b_ref,
                *out_refs, relu):
    x = x_ref[...]
    dis = dc_ref[:, 0:1]
    tx = dc_ref[:, 1:2] * x - dis * (acc_ref[0] + acc_ref[1])
    y = (jnp.dot(x, w0_ref[...], preferred_element_type=jnp.float32)
         + jnp.dot(tx, w1_ref[...], preferred_element_type=jnp.float32)
         + b_ref[...])
    if relu:
        y = jnp.maximum(y, 0.0)
        out_refs[0][...] = y
        out_refs[1][...] = dis * y
    else:
        out_refs[0][...] = y


def _make_dense(relu):
    n_out = 2 if relu else 1
    return pl.pallas_call(
        functools.partial(_dense_body, relu=relu),
        grid=(N // BLK,),
        in_specs=[
            pl.BlockSpec((BLK, D), lambda i: (i, 0)),
            pl.BlockSpec((NC, BLK, D), lambda i: (0, i, 0)),
            pl.BlockSpec((BLK, D), lambda i: (i, 0)),
            pl.BlockSpec((D, D), lambda i: (0, 0)),
            pl.BlockSpec((D, D), lambda i: (0, 0)),
            pl.BlockSpec((1, D), lambda i: (0, 0)),
        ],
        out_specs=[pl.BlockSpec((BLK, D), lambda i: (i, 0))] * n_out,
        out_shape=[jax.ShapeDtypeStruct((N, D), jnp.float32)] * n_out,
    )


_dense_relu = _make_dense(True)
_dense_last = _make_dense(False)


def kernel(embedding, W0a, W1a, b1, W0b, W1b, b2, prop_edge_index):
    pei = prop_edge_index.astype(jnp.int32)
    row1, col1 = pei[0], pei[1]
    z1 = jnp.zeros((PAD_PT,), jnp.float32)
    z2 = jnp.zeros((ROWS_PT, D), jnp.float32)

    cs_p = _deg_kernel(row1, col1, z1)
    dc, xp1 = _prep_call(cs_p[:, :, :N, None], embedding)
    acc1 = _edge_kernel(xp1, row1, col1, z2)
    h, xp2 = _dense_relu(embedding, acc1, dc, W0a, W1a, b1.reshape(1, D))
    acc2 = _edge_kernel(xp2, row1, col1, z2)
    out, = _dense_last(h, acc2, dc, W0b, W1b, b2.reshape(1, D))
    return out


# speedup vs baseline: 1.0657x; 1.0440x over previous
import functools

import jax
import jax.numpy as jnp
from jax import lax
from jax.experimental import pallas as pl
from jax.experimental.pallas import tpu as pltpu
from jax.experimental.pallas import tpu_sc as plsc

N = 10000
E = 320000
D = 128

NC, NS = 2, 16
NW = NC * NS
EPW = E // NW
CHUNK = 80
NCH = EPW // CHUNK
NPAD = 10240
PAD_PT = NPAD // NS
ROWS_PT = PAD_PT

ECHUNK = 80
ENCH = EPW // ECHUNK
ENBUF = 4
GOFF = 1
SOFF = 3
DNBUF = 5
DCHUNK = 128
NCHD = EPW // DCHUNK
DTAIL = EPW - NCHD * DCHUNK

_MESH = plsc.VectorSubcoreMesh(
    core_axis_name="c", subcore_axis_name="s", num_cores=NC, num_subcores=NS)


def _worker_id():
    return lax.axis_index("s") * NC + lax.axis_index("c")


@functools.partial(
    pl.kernel,
    out_type=jax.ShapeDtypeStruct((2, NC, NPAD), jnp.float32),
    mesh=_MESH,
    scratch_types=(
        [pltpu.VMEM((DCHUNK,), jnp.int32) for _ in range(DNBUF)]
        + [pltpu.VMEM((DCHUNK,), jnp.int32) for _ in range(DNBUF)]
        + [pltpu.VMEM((DCHUNK,), jnp.float32) for _ in range(DNBUF)]
        + [pltpu.VMEM((DTAIL,), jnp.int32),
           pltpu.VMEM((DTAIL,), jnp.int32),
           pltpu.VMEM((DTAIL,), jnp.float32),
           pltpu.VMEM((DCHUNK,), jnp.float32),
           pltpu.VMEM_SHARED((NPAD,), jnp.float32),
           pltpu.VMEM_SHARED((NPAD,), jnp.float32),
           pltpu.SemaphoreType.DMA((DNBUF,)),
           pltpu.SemaphoreType.DMA((DNBUF,)),
           pltpu.SemaphoreType.DMA((DNBUF,))]
    ),
)
def _deg_kernel(row1, col1, z1, cs_out, *rest):
    rowf = rest[:DNBUF]
    colf = rest[DNBUF:2 * DNBUF]
    slf = rest[2 * DNBUF:3 * DNBUF]
    rowt, colt, slt, onesv, cnt_acc, sl_acc, isem, s1, s2 = rest[3 * DNBUF:]
    cid = lax.axis_index("c")
    sid = lax.axis_index("s")
    wid = _worker_id()
    base = wid * EPW
    pltpu.sync_copy(z1, cnt_acc.at[pl.ds(sid * PAD_PT, PAD_PT)])
    pltpu.sync_copy(z1, sl_acc.at[pl.ds(sid * PAD_PT, PAD_PT)])
    for i in range(DCHUNK // 16):
        onesv[pl.ds(16 * i, 16)] = jnp.full((16,), 1.0, jnp.float32)
    plsc.subcore_barrier()

    @pl.loop(0, (NCHD + 2 + DNBUF - 1) // DNBUF)
    def _outer(g):
        for b in range(DNBUF):
            j = g * DNBUF + b

            @pl.when((j >= DNBUF) & (j < NCHD))
            def _reuse_wait():
                pltpu.make_async_copy(
                    onesv, cnt_acc.at[rowf[b]], s1.at[b]).wait()
                pltpu.make_async_copy(
                    slf[b], sl_acc.at[rowf[b]], s2.at[b]).wait()

            @pl.when(j < NCHD)
            def _idx_load():
                off = base + j * DCHUNK
                pltpu.async_copy(row1.at[pl.ds(off, DCHUNK)], rowf[b],
                                 isem.at[b])
                pltpu.async_copy(col1.at[pl.ds(off, DCHUNK)], colf[b],
                                 isem.at[b])

            b2 = (b - 2) % DNBUF
            j2 = j - 2

            @pl.when((j2 >= 0) & (j2 < NCHD))
            def _scatter():
                off2 = base + j2 * DCHUNK
                pltpu.make_async_copy(row1.at[pl.ds(off2, DCHUNK)], rowf[b2],
                                      isem.at[b2]).wait()
                pltpu.make_async_copy(col1.at[pl.ds(off2, DCHUNK)], colf[b2],
                                      isem.at[b2]).wait()
                for i in range(DCHUNK // 16):
                    r = rowf[b2][pl.ds(16 * i, 16)]
                    c = colf[b2][pl.ds(16 * i, 16)]
                    slf[b2][pl.ds(16 * i, 16)] = jnp.where(
                        r == c, jnp.full((16,), 1.0, jnp.float32),
                        jnp.zeros((16,), jnp.float32))
                pltpu.async_copy(onesv, cnt_acc.at[rowf[b2]], s1.at[b2],
                                 add=True)
                pltpu.async_copy(slf[b2], sl_acc.at[rowf[b2]], s2.at[b2],
                                 add=True)

    toff = base + NCHD * DCHUNK
    pltpu.sync_copy(row1.at[pl.ds(toff, DTAIL)], rowt)
    pltpu.sync_copy(col1.at[pl.ds(toff, DTAIL)], colt)
    for i in range(DTAIL // 16):
        r = rowt[pl.ds(16 * i, 16)]
        c = colt[pl.ds(16 * i, 16)]
        slt[pl.ds(16 * i, 16)] = jnp.where(
            r == c, jnp.full((16,), 1.0, jnp.float32),
            jnp.zeros((16,), jnp.float32))
    pltpu.sync_copy(onesv.at[pl.ds(0, DTAIL)], cnt_acc.at[rowt], add=True)
    pltpu.sync_copy(slt, sl_acc.at[rowt], add=True)
    for b in range(DNBUF):
        pltpu.make_async_copy(onesv, cnt_acc.at[rowf[b]], s1.at[b]).wait()
        pltpu.make_async_copy(slf[b], sl_acc.at[rowf[b]], s2.at[b]).wait()
    plsc.subcore_barrier()
    pltpu.sync_copy(cnt_acc.at[pl.ds(sid * PAD_PT, PAD_PT)],
                    cs_out.at[0, cid, pl.ds(sid * PAD_PT, PAD_PT)])
    pltpu.sync_copy(sl_acc.at[pl.ds(sid * PAD_PT, PAD_PT)],
                    cs_out.at[1, cid, pl.ds(sid * PAD_PT, PAD_PT)])


@functools.partial(
    pl.kernel,
    out_type=jax.ShapeDtypeStruct((NC, NPAD, D), jnp.float32),
    mesh=_MESH,
    scratch_types=(
        [pltpu.VMEM((ECHUNK,), jnp.int32) for _ in range(ENBUF)]
        + [pltpu.VMEM((ECHUNK,), jnp.int32) for _ in range(ENBUF)]
        + [pltpu.VMEM((ECHUNK, D), jnp.float32) for _ in range(ENBUF)]
        + [pltpu.VMEM_SHARED((NPAD, D), jnp.float32),
           pltpu.SemaphoreType.DMA((ENBUF,)),
           pltpu.SemaphoreType.DMA((ENBUF,)),
           pltpu.SemaphoreType.DMA((ENBUF,))]
    ),
)
def _edge_kernel(xp, row1, col1, z2, out, *rest):
    rowf = rest[:ENBUF]
    colf = rest[ENBUF:2 * ENBUF]
    xbs = rest[2 * ENBUF:3 * ENBUF]
    acc, isem, gsem, ssem = rest[3 * ENBUF:]
    cid = lax.axis_index("c")
    sid = lax.axis_index("s")
    wid = _worker_id()
    base = wid * EPW
    pltpu.sync_copy(z2, acc.at[pl.ds(sid * ROWS_PT, ROWS_PT)])
    plsc.subcore_barrier()

    @pl.loop(0, (ENCH + SOFF + ENBUF - 1) // ENBUF)
    def _outer(g):
        for b in range(ENBUF):
            j = g * ENBUF + b

            @pl.when((j >= ENBUF) & (j < ENCH))
            def _reuse_wait():
                pltpu.make_async_copy(
                    xbs[b], acc.at[colf[b]], ssem.at[b]).wait()

            @pl.when(j < ENCH)
            def _idx_load():
                off = base + j * ECHUNK
                pltpu.async_copy(row1.at[pl.ds(off, ECHUNK)], rowf[b],
                                 isem.at[b])
                pltpu.async_copy(col1.at[pl.ds(off, ECHUNK)], colf[b],
                                 isem.at[b])

            b2 = (b - GOFF) % ENBUF
            j2 = j - GOFF

            @pl.when((j2 >= 0) & (j2 < ENCH))
            def _gather():
                off2 = base + j2 * ECHUNK
                pltpu.make_async_copy(row1.at[pl.ds(off2, ECHUNK)], rowf[b2],
                                      isem.at[b2]).wait()
                pltpu.make_async_copy(col1.at[pl.ds(off2, ECHUNK)], colf[b2],
                                      isem.at[b2]).wait()
                pltpu.async_copy(xp.at[rowf[b2]], xbs[b2], gsem.at[b2])

            b3 = (b - SOFF) % ENBUF
            j3 = j - SOFF

            @pl.when((j3 >= 0) & (j3 < ENCH))
            def _scatter():
                pltpu.make_async_copy(xp.at[rowf[b3]], xbs[b3],
                                      gsem.at[b3]).wait()
                pltpu.async_copy(xbs[b3], acc.at[colf[b3]], ssem.at[b3],
                                 add=True)

    for b in range(ENBUF):
        pltpu.make_async_copy(xbs[b], acc.at[colf[b]], ssem.at[b]).wait()
    plsc.subcore_barrier()
    pltpu.sync_copy(acc.at[pl.ds(sid * ROWS_PT, ROWS_PT)],
                    out.at[cid, pl.ds(sid * ROWS_PT, ROWS_PT)])


BLK = 1000


BLKP = 1280


def _prep_body(cs_ref, x_ref, dc_ref, xp_ref):
    cnt = (cs_ref[0, 0] + cs_ref[0, 1]).reshape(BLKP, 1)
    sl = (cs_ref[1, 0] + cs_ref[1, 1]).reshape(BLKP, 1)
    deg = cnt - sl
    dis = jnp.where(deg > 0, lax.rsqrt(jnp.maximum(deg, 1e-12)), 0.0)
    corr = sl * dis * dis
    dc_ref[...] = jnp.concatenate(
        [dis, corr, jnp.zeros((BLKP, D - 2), jnp.float32)], axis=1)
    xp_ref[...] = dis * x_ref[...]


_prep_call = pl.pallas_call(
    _prep_body,
    grid=(NPAD // BLKP,),
    in_specs=[
        pl.BlockSpec((2, NC, BLKP), lambda i: (0, 0, i)),
        pl.BlockSpec((BLKP, D), lambda i: (i, 0)),
    ],
    out_specs=[
        pl.BlockSpec((BLKP, D), lambda i: (i, 0)),
        pl.BlockSpec((BLKP, D), lambda i: (i, 0)),
    ],
    out_shape=[
        jax.ShapeDtypeStruct((N, D), jnp.float32),
        jax.ShapeDtypeStruct((N, D), jnp.float32),
    ],
)


def _dense_body(x_ref, acc_ref, dc_ref, w0_ref, w1_ref, b_ref,
                *out_refs, relu):
    x = x_ref[...]
    dis = dc_ref[:, 0:1]
    tx = dc_ref[:, 1:2] * x - dis * (acc_ref[0] + acc_ref[1])
    y = (jnp.dot(x, w0_ref[...], preferred_element_type=jnp.float32)
         + jnp.dot(tx, w1_ref[...], preferred_element_type=jnp.float32)
         + b_ref[...])
    if relu:
        y = jnp.maximum(y, 0.0)
        out_refs[0][...] = y
        out_refs[1][...] = dis * y
    else:
        out_refs[0][...] = y


def _make_dense(relu):
    n_out = 2 if relu else 1
    return pl.pallas_call(
        functools.partial(_dense_body, relu=relu),
        grid=(N // BLK,),
        in_specs=[
            pl.BlockSpec((BLK, D), lambda i: (i, 0)),
            pl.BlockSpec((NC, BLK, D), lambda i: (0, i, 0)),
            pl.BlockSpec((BLK, D), lambda i: (i, 0)),
            pl.BlockSpec((D, D), lambda i: (0, 0)),
            pl.BlockSpec((D, D), lambda i: (0, 0)),
            pl.BlockSpec((1, D), lambda i: (0, 0)),
        ],
        out_specs=[pl.BlockSpec((BLK, D), lambda i: (i, 0))] * n_out,
        out_shape=[jax.ShapeDtypeStruct((N, D), jnp.float32)] * n_out,
    )


_dense_relu = _make_dense(True)
_dense_last = _make_dense(False)


def kernel(embedding, W0a, W1a, b1, W0b, W1b, b2, prop_edge_index):
    pei = prop_edge_index.astype(jnp.int32)
    row1, col1 = pei[0], pei[1]
    z1 = jnp.zeros((PAD_PT,), jnp.float32)
    z2 = jnp.zeros((ROWS_PT, D), jnp.float32)

    cs_p = _deg_kernel(row1, col1, z1)
    dc, xp1 = _prep_call(cs_p, embedding)
    acc1 = _edge_kernel(xp1, row1, col1, z2)
    h, xp2 = _dense_relu(embedding, acc1, dc, W0a, W1a, b1.reshape(1, D))
    acc2 = _edge_kernel(xp2, row1, col1, z2)
    out, = _dense_last(h, acc2, dc, W0b, W1b, b2.reshape(1, D))
    return out


# speedup vs baseline: 1.0828x; 1.0161x over previous
import functools

import jax
import jax.numpy as jnp
from jax import lax
from jax.experimental import pallas as pl
from jax.experimental.pallas import tpu as pltpu
from jax.experimental.pallas import tpu_sc as plsc

N = 10000
E = 320000
D = 128

NC, NS = 2, 16
NW = NC * NS
EPW = E // NW
CHUNK = 80
NCH = EPW // CHUNK
NPAD = 10240
PAD_PT = NPAD // NS
ROWS_PT = PAD_PT

ECHUNK = 80
ENCH = EPW // ECHUNK
ENBUF = 4
GOFF = 1
SOFF = 3
DNBUF = 5
DCHUNK = 128
NCHD = EPW // DCHUNK
DTAIL = EPW - NCHD * DCHUNK

_MESH = plsc.VectorSubcoreMesh(
    core_axis_name="c", subcore_axis_name="s", num_cores=NC, num_subcores=NS)


def _worker_id():
    return lax.axis_index("s") * NC + lax.axis_index("c")


@functools.partial(
    pl.kernel,
    out_type=jax.ShapeDtypeStruct((2, NC, NPAD), jnp.float32),
    mesh=_MESH,
    scratch_types=(
        [pltpu.VMEM((DCHUNK,), jnp.int32) for _ in range(DNBUF)]
        + [pltpu.VMEM((DCHUNK,), jnp.int32) for _ in range(DNBUF)]
        + [pltpu.VMEM((DCHUNK,), jnp.float32) for _ in range(DNBUF)]
        + [pltpu.VMEM((DTAIL,), jnp.int32),
           pltpu.VMEM((DTAIL,), jnp.int32),
           pltpu.VMEM((DTAIL,), jnp.float32),
           pltpu.VMEM((DCHUNK,), jnp.float32),
           pltpu.VMEM_SHARED((NPAD,), jnp.float32),
           pltpu.VMEM_SHARED((NPAD,), jnp.float32),
           pltpu.SemaphoreType.DMA((DNBUF,)),
           pltpu.SemaphoreType.DMA((DNBUF,)),
           pltpu.SemaphoreType.DMA((DNBUF,))]
    ),
)
def _deg_kernel(row1, col1, z1, cs_out, *rest):
    rowf = rest[:DNBUF]
    colf = rest[DNBUF:2 * DNBUF]
    slf = rest[2 * DNBUF:3 * DNBUF]
    rowt, colt, slt, onesv, cnt_acc, sl_acc, isem, s1, s2 = rest[3 * DNBUF:]
    cid = lax.axis_index("c")
    sid = lax.axis_index("s")
    wid = _worker_id()
    base = wid * EPW
    pltpu.sync_copy(z1, cnt_acc.at[pl.ds(sid * PAD_PT, PAD_PT)])
    pltpu.sync_copy(z1, sl_acc.at[pl.ds(sid * PAD_PT, PAD_PT)])
    for i in range(DCHUNK // 16):
        onesv[pl.ds(16 * i, 16)] = jnp.full((16,), 1.0, jnp.float32)
    plsc.subcore_barrier()

    @pl.loop(0, (NCHD + 2 + DNBUF - 1) // DNBUF)
    def _outer(g):
        for b in range(DNBUF):
            j = g * DNBUF + b

            @pl.when((j >= DNBUF) & (j < NCHD))
            def _reuse_wait():
                pltpu.make_async_copy(
                    onesv, cnt_acc.at[rowf[b]], s1.at[b]).wait()
                pltpu.make_async_copy(
                    slf[b], sl_acc.at[rowf[b]], s2.at[b]).wait()

            @pl.when(j < NCHD)
            def _idx_load():
                off = base + j * DCHUNK
                pltpu.async_copy(row1.at[pl.ds(off, DCHUNK)], rowf[b],
                                 isem.at[b])
                pltpu.async_copy(col1.at[pl.ds(off, DCHUNK)], colf[b],
                                 isem.at[b])

            b2 = (b - 2) % DNBUF
            j2 = j - 2

            @pl.when((j2 >= 0) & (j2 < NCHD))
            def _scatter():
                off2 = base + j2 * DCHUNK
                pltpu.make_async_copy(row1.at[pl.ds(off2, DCHUNK)], rowf[b2],
                                      isem.at[b2]).wait()
                pltpu.make_async_copy(col1.at[pl.ds(off2, DCHUNK)], colf[b2],
                                      isem.at[b2]).wait()
                for i in range(DCHUNK // 16):
                    r = rowf[b2][pl.ds(16 * i, 16)]
                    c = colf[b2][pl.ds(16 * i, 16)]
                    slf[b2][pl.ds(16 * i, 16)] = jnp.where(
                        r == c, jnp.full((16,), 1.0, jnp.float32),
                        jnp.zeros((16,), jnp.float32))
                pltpu.async_copy(onesv, cnt_acc.at[rowf[b2]], s1.at[b2],
                                 add=True)
                pltpu.async_copy(slf[b2], sl_acc.at[rowf[b2]], s2.at[b2],
                                 add=True)

    toff = base + NCHD * DCHUNK
    pltpu.sync_copy(row1.at[pl.ds(toff, DTAIL)], rowt)
    pltpu.sync_copy(col1.at[pl.ds(toff, DTAIL)], colt)
    for i in range(DTAIL // 16):
        r = rowt[pl.ds(16 * i, 16)]
        c = colt[pl.ds(16 * i, 16)]
        slt[pl.ds(16 * i, 16)] = jnp.where(
            r == c, jnp.full((16,), 1.0, jnp.float32),
            jnp.zeros((16,), jnp.float32))
    pltpu.sync_copy(onesv.at[pl.ds(0, DTAIL)], cnt_acc.at[rowt], add=True)
    pltpu.sync_copy(slt, sl_acc.at[rowt], add=True)
    for b in range(DNBUF):
        pltpu.make_async_copy(onesv, cnt_acc.at[rowf[b]], s1.at[b]).wait()
        pltpu.make_async_copy(slf[b], sl_acc.at[rowf[b]], s2.at[b]).wait()
    plsc.subcore_barrier()
    pltpu.sync_copy(cnt_acc.at[pl.ds(sid * PAD_PT, PAD_PT)],
                    cs_out.at[0, cid, pl.ds(sid * PAD_PT, PAD_PT)])
    pltpu.sync_copy(sl_acc.at[pl.ds(sid * PAD_PT, PAD_PT)],
                    cs_out.at[1, cid, pl.ds(sid * PAD_PT, PAD_PT)])


@functools.partial(
    pl.kernel,
    out_type=jax.ShapeDtypeStruct((NC, NPAD, D), jnp.float32),
    mesh=_MESH,
    scratch_types=(
        [pltpu.VMEM((ECHUNK,), jnp.int32) for _ in range(ENBUF)]
        + [pltpu.VMEM((ECHUNK,), jnp.int32) for _ in range(ENBUF)]
        + [pltpu.VMEM((ECHUNK, D), jnp.float32) for _ in range(ENBUF)]
        + [pltpu.VMEM_SHARED((NPAD, D), jnp.float32),
           pltpu.SemaphoreType.DMA((ENBUF,)),
           pltpu.SemaphoreType.DMA((ENBUF,)),
           pltpu.SemaphoreType.DMA((ENBUF,))]
    ),
)
def _edge_kernel(xp, row1, col1, z2, out, *rest):
    rowf = rest[:ENBUF]
    colf = rest[ENBUF:2 * ENBUF]
    xbs = rest[2 * ENBUF:3 * ENBUF]
    acc, isem, gsem, ssem = rest[3 * ENBUF:]
    cid = lax.axis_index("c")
    sid = lax.axis_index("s")
    wid = _worker_id()
    base = wid * EPW
    pltpu.sync_copy(z2, acc.at[pl.ds(sid * ROWS_PT, ROWS_PT)])
    plsc.subcore_barrier()

    @pl.loop(0, (ENCH + SOFF + ENBUF - 1) // ENBUF)
    def _outer(g):
        for b in range(ENBUF):
            j = g * ENBUF + b

            @pl.when((j >= ENBUF) & (j < ENCH))
            def _reuse_wait():
                pltpu.make_async_copy(
                    xbs[b], acc.at[colf[b]], ssem.at[b]).wait()

            @pl.when(j < ENCH)
            def _idx_load():
                off = base + j * ECHUNK
                pltpu.async_copy(row1.at[pl.ds(off, ECHUNK)], rowf[b],
                                 isem.at[b])
                pltpu.async_copy(col1.at[pl.ds(off, ECHUNK)], colf[b],
                                 isem.at[b])

            b2 = (b - GOFF) % ENBUF
            j2 = j - GOFF

            @pl.when((j2 >= 0) & (j2 < ENCH))
            def _gather():
                off2 = base + j2 * ECHUNK
                pltpu.make_async_copy(row1.at[pl.ds(off2, ECHUNK)], rowf[b2],
                                      isem.at[b2]).wait()
                pltpu.make_async_copy(col1.at[pl.ds(off2, ECHUNK)], colf[b2],
                                      isem.at[b2]).wait()
                pltpu.async_copy(xp.at[rowf[b2]], xbs[b2], gsem.at[b2])

            b3 = (b - SOFF) % ENBUF
            j3 = j - SOFF

            @pl.when((j3 >= 0) & (j3 < ENCH))
            def _scatter():
                pltpu.make_async_copy(xp.at[rowf[b3]], xbs[b3],
                                      gsem.at[b3]).wait()
                pltpu.async_copy(xbs[b3], acc.at[colf[b3]], ssem.at[b3],
                                 add=True)

    for b in range(ENBUF):
        pltpu.make_async_copy(xbs[b], acc.at[colf[b]], ssem.at[b]).wait()
    plsc.subcore_barrier()
    pltpu.sync_copy(acc.at[pl.ds(sid * ROWS_PT, ROWS_PT)],
                    out.at[cid, pl.ds(sid * ROWS_PT, ROWS_PT)])


BLK = 2000


BLKP = 1280


def _prep_body(cs_ref, x_ref, dc_ref, xp_ref):
    cnt = (cs_ref[0, 0] + cs_ref[0, 1]).reshape(BLKP, 1)
    sl = (cs_ref[1, 0] + cs_ref[1, 1]).reshape(BLKP, 1)
    deg = cnt - sl
    dis = jnp.where(deg > 0, lax.rsqrt(jnp.maximum(deg, 1e-12)), 0.0)
    corr = sl * dis * dis
    dc_ref[...] = jnp.concatenate(
        [dis, corr, jnp.zeros((BLKP, D - 2), jnp.float32)], axis=1)
    xp_ref[...] = dis * x_ref[...]


_prep_call = pl.pallas_call(
    _prep_body,
    grid=(NPAD // BLKP,),
    in_specs=[
        pl.BlockSpec((2, NC, BLKP), lambda i: (0, 0, i)),
        pl.BlockSpec((BLKP, D), lambda i: (i, 0)),
    ],
    out_specs=[
        pl.BlockSpec((BLKP, D), lambda i: (i, 0)),
        pl.BlockSpec((BLKP, D), lambda i: (i, 0)),
    ],
    out_shape=[
        jax.ShapeDtypeStruct((N, D), jnp.float32),
        jax.ShapeDtypeStruct((N, D), jnp.float32),
    ],
)


def _dense_body(x_ref, acc_ref, dc_ref, w0_ref, w1_ref, b_ref,
                *out_refs, relu):
    x = x_ref[...]
    dis = dc_ref[:, 0:1]
    tx = dc_ref[:, 1:2] * x - dis * (acc_ref[0] + acc_ref[1])
    y = (jnp.dot(x, w0_ref[...], preferred_element_type=jnp.float32)
         + jnp.dot(tx, w1_ref[...], preferred_element_type=jnp.float32)
         + b_ref[...])
    if relu:
        y = jnp.maximum(y, 0.0)
        out_refs[0][...] = y
        out_refs[1][...] = dis * y
    else:
        out_refs[0][...] = y


def _make_dense(relu):
    n_out = 2 if relu else 1
    return pl.pallas_call(
        functools.partial(_dense_body, relu=relu),
        grid=(N // BLK,),
        in_specs=[
            pl.BlockSpec((BLK, D), lambda i: (i, 0)),
            pl.BlockSpec((NC, BLK, D), lambda i: (0, i, 0)),
            pl.BlockSpec((BLK, D), lambda i: (i, 0)),
            pl.BlockSpec((D, D), lambda i: (0, 0)),
            pl.BlockSpec((D, D), lambda i: (0, 0)),
            pl.BlockSpec((1, D), lambda i: (0, 0)),
        ],
        out_specs=[pl.BlockSpec((BLK, D), lambda i: (i, 0))] * n_out,
        out_shape=[jax.ShapeDtypeStruct((N, D), jnp.float32)] * n_out,
    )


_dense_relu = _make_dense(True)
_dense_last = _make_dense(False)


def kernel(embedding, W0a, W1a, b1, W0b, W1b, b2, prop_edge_index):
    pei = prop_edge_index.astype(jnp.int32)
    row1, col1 = pei[0], pei[1]
    z1 = jnp.zeros((PAD_PT,), jnp.float32)
    z2 = jnp.zeros((ROWS_PT, D), jnp.float32)

    cs_p = _deg_kernel(row1, col1, z1)
    dc, xp1 = _prep_call(cs_p, embedding)
    acc1 = _edge_kernel(xp1, row1, col1, z2)
    h, xp2 = _dense_relu(embedding, acc1, dc, W0a, W1a, b1.reshape(1, D))
    acc2 = _edge_kernel(xp2, row1, col1, z2)
    out, = _dense_last(h, acc2, dc, W0b, W1b, b2.reshape(1, D))
    return out


# speedup vs baseline: 1.1412x; 1.0539x over previous
import functools

import jax
import jax.numpy as jnp
from jax import lax
from jax.experimental import pallas as pl
from jax.experimental.pallas import tpu as pltpu
from jax.experimental.pallas import tpu_sc as plsc

N = 10000
E = 320000
D = 128

NC, NS = 2, 16
NW = NC * NS
EPW = E // NW
CHUNK = 80
NCH = EPW // CHUNK
NPAD = 10240
PAD_PT = NPAD // NS
ROWS_PT = PAD_PT

ECHUNK = 80
ENCH = EPW // ECHUNK
ENBUF = 4
GOFF = 1
SOFF = 3
DNBUF = 5
DCHUNK = 128
NCHD = EPW // DCHUNK
DTAIL = EPW - NCHD * DCHUNK

_MESH = plsc.VectorSubcoreMesh(
    core_axis_name="c", subcore_axis_name="s", num_cores=NC, num_subcores=NS)


def _worker_id():
    return lax.axis_index("s") * NC + lax.axis_index("c")


NBLK_ALL = E // DCHUNK


@functools.partial(
    pl.kernel,
    out_type=[
        jax.ShapeDtypeStruct((2, NC, NPAD), jnp.float32),
        jax.ShapeDtypeStruct((E,), jnp.int32),
        jax.ShapeDtypeStruct((E,), jnp.int32),
    ],
    mesh=_MESH,
    scratch_types=(
        [pltpu.VMEM((2, DCHUNK), jnp.int32) for _ in range(DNBUF)]
        + [pltpu.VMEM((DCHUNK,), jnp.float32) for _ in range(DNBUF)]
        + [pltpu.VMEM((DCHUNK,), jnp.float32),
           pltpu.VMEM_SHARED((NPAD,), jnp.float32),
           pltpu.VMEM_SHARED((NPAD,), jnp.float32),
           pltpu.SemaphoreType.DMA((DNBUF,)),
           pltpu.SemaphoreType.DMA((DNBUF,)),
           pltpu.SemaphoreType.DMA((DNBUF,)),
           pltpu.SemaphoreType.DMA((DNBUF,))]
    ),
)
def _deg_kernel(pei, z1, cs_out, row_out, col_out, *rest):
    rcf = rest[:DNBUF]
    slf = rest[DNBUF:2 * DNBUF]
    onesv, cnt_acc, sl_acc, isem, s1, s2, wsem = rest[2 * DNBUF:]
    cid = lax.axis_index("c")
    sid = lax.axis_index("s")
    wid = _worker_id()
    nblk = 78 + jnp.where(wid < NBLK_ALL - 78 * NW, 1, 0)
    pltpu.sync_copy(z1, cnt_acc.at[pl.ds(sid * PAD_PT, PAD_PT)])
    pltpu.sync_copy(z1, sl_acc.at[pl.ds(sid * PAD_PT, PAD_PT)])
    for i in range(DCHUNK // 16):
        onesv[pl.ds(16 * i, 16)] = jnp.full((16,), 1.0, jnp.float32)
    plsc.subcore_barrier()

    @pl.loop(0, (nblk + 2 + DNBUF - 1) // DNBUF)
    def _outer(g):
        for b in range(DNBUF):
            j = g * DNBUF + b

            @pl.when((j >= DNBUF) & (j < nblk))
            def _reuse_wait():
                pltpu.make_async_copy(
                    onesv, cnt_acc.at[rcf[b].at[0]], s1.at[b]).wait()
                pltpu.make_async_copy(
                    slf[b], sl_acc.at[rcf[b].at[0]], s2.at[b]).wait()
                off_old = (wid + (j - DNBUF) * NW) * DCHUNK
                pltpu.make_async_copy(
                    rcf[b].at[0], row_out.at[pl.ds(off_old, DCHUNK)],
                    wsem.at[b]).wait()
                pltpu.make_async_copy(
                    rcf[b].at[1], col_out.at[pl.ds(off_old, DCHUNK)],
                    wsem.at[b]).wait()

            @pl.when(j < nblk)
            def _idx_load():
                off = (wid + j * NW) * DCHUNK
                pltpu.async_copy(pei.at[:, pl.ds(off, DCHUNK)], rcf[b],
                                 isem.at[b])

            b2 = (b - 2) % DNBUF
            j2 = j - 2

            @pl.when((j2 >= 0) & (j2 < nblk))
            def _scatter():
                off2 = (wid + j2 * NW) * DCHUNK
                pltpu.make_async_copy(pei.at[:, pl.ds(off2, DCHUNK)], rcf[b2],
                                      isem.at[b2]).wait()
                for i in range(DCHUNK // 16):
                    r = rcf[b2][0, pl.ds(16 * i, 16)]
                    c = rcf[b2][1, pl.ds(16 * i, 16)]
                    slf[b2][pl.ds(16 * i, 16)] = jnp.where(
                        r == c, jnp.full((16,), 1.0, jnp.float32),
                        jnp.zeros((16,), jnp.float32))
                pltpu.async_copy(onesv, cnt_acc.at[rcf[b2].at[0]], s1.at[b2],
                                 add=True)
                pltpu.async_copy(slf[b2], sl_acc.at[rcf[b2].at[0]], s2.at[b2],
                                 add=True)
                pltpu.async_copy(rcf[b2].at[0],
                                 row_out.at[pl.ds(off2, DCHUNK)], wsem.at[b2])
                pltpu.async_copy(rcf[b2].at[1],
                                 col_out.at[pl.ds(off2, DCHUNK)], wsem.at[b2])

    for b in range(DNBUF):
        pltpu.make_async_copy(onesv, cnt_acc.at[rcf[b].at[0]], s1.at[b]).wait()
        pltpu.make_async_copy(slf[b], sl_acc.at[rcf[b].at[0]], s2.at[b]).wait()
        pltpu.make_async_copy(rcf[b].at[0], row_out.at[pl.ds(0, DCHUNK)],
                              wsem.at[b]).wait()
        pltpu.make_async_copy(rcf[b].at[1], col_out.at[pl.ds(0, DCHUNK)],
                              wsem.at[b]).wait()
    plsc.subcore_barrier()
    pltpu.sync_copy(cnt_acc.at[pl.ds(sid * PAD_PT, PAD_PT)],
                    cs_out.at[0, cid, pl.ds(sid * PAD_PT, PAD_PT)])
    pltpu.sync_copy(sl_acc.at[pl.ds(sid * PAD_PT, PAD_PT)],
                    cs_out.at[1, cid, pl.ds(sid * PAD_PT, PAD_PT)])


@functools.partial(
    pl.kernel,
    out_type=jax.ShapeDtypeStruct((NC, NPAD, D), jnp.float32),
    mesh=_MESH,
    scratch_types=(
        [pltpu.VMEM((ECHUNK,), jnp.int32) for _ in range(ENBUF)]
        + [pltpu.VMEM((ECHUNK,), jnp.int32) for _ in range(ENBUF)]
        + [pltpu.VMEM((ECHUNK, D), jnp.float32) for _ in range(ENBUF)]
        + [pltpu.VMEM_SHARED((NPAD, D), jnp.float32),
           pltpu.SemaphoreType.DMA((ENBUF,)),
           pltpu.SemaphoreType.DMA((ENBUF,)),
           pltpu.SemaphoreType.DMA((ENBUF,))]
    ),
)
def _edge_kernel(xp, row1, col1, z2, out, *rest):
    rowf = rest[:ENBUF]
    colf = rest[ENBUF:2 * ENBUF]
    xbs = rest[2 * ENBUF:3 * ENBUF]
    acc, isem, gsem, ssem = rest[3 * ENBUF:]
    cid = lax.axis_index("c")
    sid = lax.axis_index("s")
    wid = _worker_id()
    base = wid * EPW
    pltpu.sync_copy(z2, acc.at[pl.ds(sid * ROWS_PT, ROWS_PT)])
    plsc.subcore_barrier()

    @pl.loop(0, (ENCH + SOFF + ENBUF - 1) // ENBUF)
    def _outer(g):
        for b in range(ENBUF):
            j = g * ENBUF + b

            @pl.when((j >= ENBUF) & (j < ENCH))
            def _reuse_wait():
                pltpu.make_async_copy(
                    xbs[b], acc.at[colf[b]], ssem.at[b]).wait()

            @pl.when(j < ENCH)
            def _idx_load():
                off = base + j * ECHUNK
                pltpu.async_copy(row1.at[pl.ds(off, ECHUNK)], rowf[b],
                                 isem.at[b])
                pltpu.async_copy(col1.at[pl.ds(off, ECHUNK)], colf[b],
                                 isem.at[b])

            b2 = (b - GOFF) % ENBUF
            j2 = j - GOFF

            @pl.when((j2 >= 0) & (j2 < ENCH))
            def _gather():
                off2 = base + j2 * ECHUNK
                pltpu.make_async_copy(row1.at[pl.ds(off2, ECHUNK)], rowf[b2],
                                      isem.at[b2]).wait()
                pltpu.make_async_copy(col1.at[pl.ds(off2, ECHUNK)], colf[b2],
                                      isem.at[b2]).wait()
                pltpu.async_copy(xp.at[rowf[b2]], xbs[b2], gsem.at[b2])

            b3 = (b - SOFF) % ENBUF
            j3 = j - SOFF

            @pl.when((j3 >= 0) & (j3 < ENCH))
            def _scatter():
                pltpu.make_async_copy(xp.at[rowf[b3]], xbs[b3],
                                      gsem.at[b3]).wait()
                pltpu.async_copy(xbs[b3], acc.at[colf[b3]], ssem.at[b3],
                                 add=True)

    for b in range(ENBUF):
        pltpu.make_async_copy(xbs[b], acc.at[colf[b]], ssem.at[b]).wait()
    plsc.subcore_barrier()
    pltpu.sync_copy(acc.at[pl.ds(sid * ROWS_PT, ROWS_PT)],
                    out.at[cid, pl.ds(sid * ROWS_PT, ROWS_PT)])


BLK = 2000


BLKP = 1280


def _prep_body(cs_ref, x_ref, dc_ref, xp_ref):
    cnt = (cs_ref[0, 0] + cs_ref[0, 1]).reshape(BLKP, 1)
    sl = (cs_ref[1, 0] + cs_ref[1, 1]).reshape(BLKP, 1)
    deg = cnt - sl
    dis = jnp.where(deg > 0, lax.rsqrt(jnp.maximum(deg, 1e-12)), 0.0)
    corr = sl * dis * dis
    dc_ref[...] = jnp.concatenate(
        [dis, corr, jnp.zeros((BLKP, D - 2), jnp.float32)], axis=1)
    xp_ref[...] = dis * x_ref[...]


_prep_call = pl.pallas_call(
    _prep_body,
    grid=(NPAD // BLKP,),
    in_specs=[
        pl.BlockSpec((2, NC, BLKP), lambda i: (0, 0, i)),
        pl.BlockSpec((BLKP, D), lambda i: (i, 0)),
    ],
    out_specs=[
        pl.BlockSpec((BLKP, D), lambda i: (i, 0)),
        pl.BlockSpec((BLKP, D), lambda i: (i, 0)),
    ],
    out_shape=[
        jax.ShapeDtypeStruct((N, D), jnp.float32),
        jax.ShapeDtypeStruct((N, D), jnp.float32),
    ],
)


def _dense_body(x_ref, acc_ref, dc_ref, w0_ref, w1_ref, b_ref,
                *out_refs, relu):
    x = x_ref[...]
    dis = dc_ref[:, 0:1]
    tx = dc_ref[:, 1:2] * x - dis * (acc_ref[0] + acc_ref[1])
    y = (jnp.dot(x, w0_ref[...], preferred_element_type=jnp.float32)
         + jnp.dot(tx, w1_ref[...], preferred_element_type=jnp.float32)
         + b_ref[...])
    if relu:
        y = jnp.maximum(y, 0.0)
        out_refs[0][...] = y
        out_refs[1][...] = dis * y
    else:
        out_refs[0][...] = y


def _make_dense(relu):
    n_out = 2 if relu else 1
    return pl.pallas_call(
        functools.partial(_dense_body, relu=relu),
        grid=(N // BLK,),
        in_specs=[
            pl.BlockSpec((BLK, D), lambda i: (i, 0)),
            pl.BlockSpec((NC, BLK, D), lambda i: (0, i, 0)),
            pl.BlockSpec((BLK, D), lambda i: (i, 0)),
            pl.BlockSpec((D, D), lambda i: (0, 0)),
            pl.BlockSpec((D, D), lambda i: (0, 0)),
            pl.BlockSpec((1, D), lambda i: (0, 0)),
        ],
        out_specs=[pl.BlockSpec((BLK, D), lambda i: (i, 0))] * n_out,
        out_shape=[jax.ShapeDtypeStruct((N, D), jnp.float32)] * n_out,
    )


_dense_relu = _make_dense(True)
_dense_last = _make_dense(False)


def kernel(embedding, W0a, W1a, b1, W0b, W1b, b2, prop_edge_index):
    pei = prop_edge_index.astype(jnp.int32)
    z1 = jnp.zeros((PAD_PT,), jnp.float32)
    z2 = jnp.zeros((ROWS_PT, D), jnp.float32)

    cs_p, row1, col1 = _deg_kernel(pei, z1)
    dc, xp1 = _prep_call(cs_p, embedding)
    acc1 = _edge_kernel(xp1, row1, col1, z2)
    h, xp2 = _dense_relu(embedding, acc1, dc, W0a, W1a, b1.reshape(1, D))
    acc2 = _edge_kernel(xp2, row1, col1, z2)
    out, = _dense_last(h, acc2, dc, W0b, W1b, b2.reshape(1, D))
    return out
